# baseline (speedup 1.0000x reference)
import functools

import jax
import jax.numpy as jnp
from jax import lax
from jax.experimental import pallas as pl
from jax.experimental.pallas import tpu as pltpu
from jax.experimental.pallas import tpu_sc as plsc

_P = 128
_ALPHA = 16
_LANES = 16
_NW = 32


def _sc_gather_call(W, tokens):
    Pdim, V = W.shape
    N = tokens.shape[0]
    rows_per = Pdim // _NW
    chunk = 4096
    n_chunks = N // chunk
    mesh = plsc.VectorSubcoreMesh(core_axis_name="c", subcore_axis_name="s",
                                  num_cores=2, num_subcores=16)

    def body(W_hbm, tok_hbm, out_hbm, wrow_v, tok_v, obuf_v,
             sem_tok, sem_row, sem_o0, sem_o1):
        cid = lax.axis_index("c")
        sid = lax.axis_index("s")
        wid = sid * 2 + cid
        pltpu.async_copy(tok_hbm, tok_v, sem_tok).wait()
        out_sems = (sem_o0, sem_o1)
        last = [None, None]
        k = 0
        for r in range(rows_per):
            p = wid * rows_per + r
            pltpu.async_copy(W_hbm.at[p], wrow_v, sem_row).wait()
            for c in range(n_chunks):
                b = k % 2
                if last[b] is not None:
                    last[b].wait()

                @plsc.parallel_loop(0, chunk // _LANES, unroll=8)
                def _gather(i, c=c, b=b):
                    idx = tok_v[pl.ds(c * chunk + i * _LANES, _LANES)]
                    obuf_v[b, pl.ds(i * _LANES, _LANES)] = plsc.load_gather(
                        wrow_v, [idx])

                last[b] = pltpu.async_copy(
                    obuf_v.at[b], out_hbm.at[p, pl.ds(c * chunk, chunk)],
                    out_sems[b])
                k += 1
        for d in last:
            if d is not None:
                d.wait()

    return pl.kernel(
        body,
        out_type=jax.ShapeDtypeStruct((Pdim, N), jnp.float32),
        mesh=mesh,
        scratch_types=[
            pltpu.VMEM((V,), jnp.float32),
            pltpu.VMEM((N,), jnp.int32),
            pltpu.VMEM((2, chunk), jnp.float32),
            pltpu.SemaphoreType.DMA,
            pltpu.SemaphoreType.DMA,
            pltpu.SemaphoreType.DMA,
            pltpu.SemaphoreType.DMA,
        ],
        compiler_params=pltpu.CompilerParams(needs_layout_passes=False),
    )(W, tokens)


def _tc_expand_call(UT, SP2, f_rep):
    Pdim, N = UT.shape
    M = _P * _ALPHA
    bn = 512
    grid = (N // bn,)

    def body(ut_ref, sp_ref, frep_ref, u_ref, fo_ref, y_ref, e_v):
        @pl.when(pl.program_id(0) == 0)
        def _build_e():
            col = lax.broadcasted_iota(jnp.int32, (Pdim, M), 1)
            row = lax.broadcasted_iota(jnp.int32, (Pdim, M), 0)
            ondiag = (col // _ALPHA) == row
            e_v[...] = jnp.where(ondiag, frep_ref[...],
                                 jnp.zeros_like(frep_ref[...]))

        u = ut_ref[...].T
        sp = sp_ref[...]
        m = u >= sp
        u_ref[...] = u
        fo_ref[...] = m.astype(jnp.float32)
        t = jnp.where(m, u, 0.0).astype(jnp.bfloat16)
        y_ref[...] = lax.dot_general(
            t, e_v[...], (((1,), (0,)), ((), ())),
            preferred_element_type=jnp.float32)

    return pl.pallas_call(
        body,
        grid=grid,
        in_specs=[
            pl.BlockSpec((Pdim, bn), lambda i: (0, i)),
            pl.BlockSpec((1, Pdim), lambda i: (0, 0)),
            pl.BlockSpec((Pdim, M), lambda i: (0, 0)),
        ],
        out_specs=[
            pl.BlockSpec((bn, Pdim), lambda i: (i, 0)),
            pl.BlockSpec((bn, Pdim), lambda i: (i, 0)),
            pl.BlockSpec((bn, M), lambda i: (i, 0)),
        ],
        out_shape=[
            jax.ShapeDtypeStruct((N, Pdim), jnp.float32),
            jax.ShapeDtypeStruct((N, Pdim), jnp.float32),
            jax.ShapeDtypeStruct((N, M), jnp.float32),
        ],
        scratch_shapes=[pltpu.VMEM((Pdim, M), jnp.bfloat16)],
        compiler_params=pltpu.CompilerParams(
            dimension_semantics=("arbitrary",)),
    )(UT, SP2, f_rep)


def kernel(input_ids, W, f, SP):
    B, T = input_ids.shape
    tokens = input_ids.reshape(-1)
    UT = _sc_gather_call(W, tokens)
    f_rep = jnp.tile(f, (1, _P)).astype(jnp.bfloat16)
    U2, F2, Y2 = _tc_expand_call(UT, SP.reshape(1, _P).astype(jnp.float32),
                                 f_rep)
    return (U2.reshape(B, T, _P), F2.reshape(B, T, _P),
            Y2.reshape(B, T, _P, _ALPHA))


# speedup vs baseline: 4.0773x; 4.0773x over previous
import functools

import jax
import jax.numpy as jnp
from jax import lax
from jax.experimental import pallas as pl
from jax.experimental.pallas import tpu as pltpu
from jax.experimental.pallas import tpu_sc as plsc

_P = 128
_ALPHA = 16
_LANES = 16
_NW = 32


def _sc_gather_call(W, tokens):
    Pdim, V = W.shape
    N = tokens.shape[0]
    rows_per = Pdim // _NW
    chunk = 4096
    n_chunks = N // chunk
    mesh = plsc.VectorSubcoreMesh(core_axis_name="c", subcore_axis_name="s",
                                  num_cores=2, num_subcores=16)

    def body(W_hbm, tok_hbm, out_hbm, wrow_v, tok_v, obuf_v,
             sem_tok, sem_row, sem_o0, sem_o1):
        cid = lax.axis_index("c")
        sid = lax.axis_index("s")
        wid = sid * 2 + cid
        pltpu.async_copy(tok_hbm, tok_v, sem_tok).wait()
        out_sems = (sem_o0, sem_o1)
        last = [None, None]
        k = 0
        for r in range(rows_per):
            p = wid * rows_per + r
            pltpu.async_copy(W_hbm.at[p], wrow_v, sem_row).wait()
            for c in range(n_chunks):
                b = k % 2
                if last[b] is not None:
                    last[b].wait()

                @plsc.parallel_loop(0, chunk // _LANES, unroll=8)
                def _gather(i, c=c, b=b):
                    idx = tok_v[pl.ds(c * chunk + i * _LANES, _LANES)]
                    obuf_v[b, pl.ds(i * _LANES, _LANES)] = plsc.load_gather(
                        wrow_v, [idx])

                last[b] = pltpu.async_copy(
                    obuf_v.at[b], out_hbm.at[p, pl.ds(c * chunk, chunk)],
                    out_sems[b])
                k += 1
        for d in last:
            if d is not None:
                d.wait()

    return pl.kernel(
        body,
        out_type=jax.ShapeDtypeStruct((Pdim, N), jnp.float32),
        mesh=mesh,
        scratch_types=[
            pltpu.VMEM((V,), jnp.float32),
            pltpu.VMEM((N,), jnp.int32),
            pltpu.VMEM((2, chunk), jnp.float32),
            pltpu.SemaphoreType.DMA,
            pltpu.SemaphoreType.DMA,
            pltpu.SemaphoreType.DMA,
            pltpu.SemaphoreType.DMA,
        ],
        compiler_params=pltpu.CompilerParams(needs_layout_passes=False),
    )(W, tokens)


def _tc_expand_call(UT, SP2, fT, B, T):
    Pdim, N = UT.shape
    M = _ALPHA * Pdim

    def body(ut_ref, sp_ref, ft_ref, u_ref, fo_ref, y_ref):
        u = ut_ref[...].T
        sp = sp_ref[...]
        msk = u >= sp
        u_ref[...] = u
        fo_ref[...] = msk.astype(jnp.float32)
        t2 = jnp.where(msk, u, 0.0)
        ft = ft_ref[...]
        pieces = [t2 * ft[a:a + 1, :] for a in range(_ALPHA)]
        y = jnp.concatenate(pieces, axis=1)
        y_ref[...] = y

    return pl.pallas_call(
        body,
        grid=(T,),
        in_specs=[
            pl.BlockSpec((Pdim, B), lambda i: (0, i)),
            pl.BlockSpec((1, Pdim), lambda i: (0, 0)),
            pl.BlockSpec((_ALPHA, Pdim), lambda i: (0, 0)),
        ],
        out_specs=[
            pl.BlockSpec((B, Pdim), lambda i: (i, 0)),
            pl.BlockSpec((B, Pdim), lambda i: (i, 0)),
            pl.BlockSpec((B, M), lambda i: (0, i)),
        ],
        out_shape=[
            jax.ShapeDtypeStruct((N, Pdim), jnp.float32),
            jax.ShapeDtypeStruct((N, Pdim), jnp.float32),
            jax.ShapeDtypeStruct((B, T * M), jnp.float32),
        ],
        compiler_params=pltpu.CompilerParams(
            dimension_semantics=("arbitrary",)),
    )(UT, SP2, fT)


def kernel(input_ids, W, f, SP):
    B, T = input_ids.shape
    tokens_m = input_ids.T.reshape(-1)
    UT = _sc_gather_call(W, tokens_m)
    Um, Fm, Y4 = _tc_expand_call(
        UT, SP.reshape(1, _P).astype(jnp.float32), f.T, B, T)
    U = jnp.transpose(Um.reshape(T, B, _P), (1, 0, 2))
    F = jnp.transpose(Fm.reshape(T, B, _P), (1, 0, 2))
    Y = jnp.transpose(Y4.reshape(B, T, _ALPHA, _P), (0, 1, 3, 2))
    return (U, F, Y)


# speedup vs baseline: 7.0214x; 1.7221x over previous
import functools

import jax
import jax.numpy as jnp
from jax import lax
from jax.experimental import pallas as pl
from jax.experimental.pallas import tpu as pltpu
from jax.experimental.pallas import tpu_sc as plsc

_P = 128
_ALPHA = 16
_LANES = 16
_NW = 32


def _sc_gather_call(W, tokens):
    Pdim, V = W.shape
    N = tokens.shape[0]
    rows_per = Pdim // _NW
    chunk = 4096
    n_chunks = N // chunk
    mesh = plsc.VectorSubcoreMesh(core_axis_name="c", subcore_axis_name="s",
                                  num_cores=2, num_subcores=16)

    def body(W_hbm, tok_hbm, out_hbm, wrow_v, tok_v, obuf_v,
             sem_tok, sem_row, sem_o0, sem_o1):
        cid = lax.axis_index("c")
        sid = lax.axis_index("s")
        wid = sid * 2 + cid
        pltpu.async_copy(tok_hbm, tok_v, sem_tok).wait()
        out_sems = (sem_o0, sem_o1)
        last = [None, None]
        k = 0
        for r in range(rows_per):
            p = wid * rows_per + r
            pltpu.async_copy(W_hbm.at[p], wrow_v, sem_row).wait()
            for c in range(n_chunks):
                b = k % 2
                if last[b] is not None:
                    last[b].wait()

                @plsc.parallel_loop(0, chunk // _LANES, unroll=8)
                def _gather(i, c=c, b=b):
                    idx = tok_v[pl.ds(c * chunk + i * _LANES, _LANES)]
                    obuf_v[b, pl.ds(i * _LANES, _LANES)] = plsc.load_gather(
                        wrow_v, [idx])

                last[b] = pltpu.async_copy(
                    obuf_v.at[b], out_hbm.at[p, pl.ds(c * chunk, chunk)],
                    out_sems[b])
                k += 1
        for d in last:
            if d is not None:
                d.wait()

    return pl.kernel(
        body,
        out_type=jax.ShapeDtypeStruct((Pdim, N), jnp.float32),
        mesh=mesh,
        scratch_types=[
            pltpu.VMEM((V,), jnp.float32),
            pltpu.VMEM((N,), jnp.int32),
            pltpu.VMEM((2, chunk), jnp.float32),
            pltpu.SemaphoreType.DMA,
            pltpu.SemaphoreType.DMA,
            pltpu.SemaphoreType.DMA,
            pltpu.SemaphoreType.DMA,
        ],
        compiler_params=pltpu.CompilerParams(needs_layout_passes=False),
    )(W, tokens)


def _tc_expand_call(UT, SP2, fT, B, T):
    Pdim, N = UT.shape
    M = _ALPHA * Pdim

    def body(ut_ref, sp_ref, ft_ref, u_ref, fo_ref, y_ref):
        u = ut_ref[...].T
        sp = sp_ref[...]
        msk = u >= sp
        u_ref[...] = u
        fo_ref[...] = msk.astype(jnp.float32)
        t2 = jnp.where(msk, u, 0.0)
        ft = ft_ref[...]
        y = (jnp.broadcast_to(t2[:, None, :], (B, _ALPHA, Pdim))
             * jnp.broadcast_to(ft[None, :, :], (B, _ALPHA, Pdim)))
        y_ref[...] = y

    return pl.pallas_call(
        body,
        grid=(T,),
        in_specs=[
            pl.BlockSpec((Pdim, B), lambda i: (0, i)),
            pl.BlockSpec((1, Pdim), lambda i: (0, 0)),
            pl.BlockSpec((_ALPHA, Pdim), lambda i: (0, 0)),
        ],
        out_specs=[
            pl.BlockSpec((B, Pdim), lambda i: (i, 0)),
            pl.BlockSpec((B, Pdim), lambda i: (i, 0)),
            pl.BlockSpec((B, _ALPHA, Pdim), lambda i: (0, i, 0)),
        ],
        out_shape=[
            jax.ShapeDtypeStruct((N, Pdim), jnp.float32),
            jax.ShapeDtypeStruct((N, Pdim), jnp.float32),
            jax.ShapeDtypeStruct((B, T * _ALPHA, Pdim), jnp.float32),
        ],
        compiler_params=pltpu.CompilerParams(
            dimension_semantics=("arbitrary",)),
    )(UT, SP2, fT)


def kernel(input_ids, W, f, SP):
    B, T = input_ids.shape
    tokens_m = input_ids.T.reshape(-1)
    UT = _sc_gather_call(W, tokens_m)
    Um, Fm, Y4 = _tc_expand_call(
        UT, SP.reshape(1, _P).astype(jnp.float32), f.T, B, T)
    U = jnp.transpose(Um.reshape(T, B, _P), (1, 0, 2))
    F = jnp.transpose(Fm.reshape(T, B, _P), (1, 0, 2))
    Y = jnp.transpose(Y4.reshape(B, T, _ALPHA, _P), (0, 1, 3, 2))
    return (U, F, Y)
